# Initial kernel scaffold; baseline (speedup 1.0000x reference)
import jax
import jax.numpy as jnp
from jax.experimental import pallas as pl


def kernel(x, edge_index, edge_attr, w_gate, w_noise, expert_W0, expert_b0, expert_W1, expert_b1):
    raise NotImplementedError("write your pallas kernel here")


# speedup vs baseline: 5.2938x; 5.2938x over previous
import numpy as np
import jax
import jax.numpy as jnp
from jax.experimental import pallas as pl

_N_NODES = 10000
_N_EDGES = 320000
_IN_DIM = 128
_EDGE_DIM = 4
_HIDDEN = 32
_NUM_EXPERTS = 8
_TOP_K = 2
_K_LIST = [0.3, 0.4, 0.5, 0.6, 0.7, 0.8, 0.9, 1.0]
_LOSS_COEF = 0.01

_K_TABLE = np.stack(
    [
        np.maximum(
            np.round(np.arange(_N_EDGES + 1).astype(np.float64) * k).astype(np.int64),
            1,
        )
        for k in _K_LIST
    ],
    axis=1,
).astype(np.int32)


def _cv_squared(v):
    eps = 1e-10
    v = v.astype(jnp.float32)
    return jnp.var(v, ddof=1) / (jnp.mean(v) ** 2 + eps)


def kernel(x, edge_index, edge_attr, w_gate, w_noise, expert_W0, expert_b0,
           expert_W1, expert_b1):
    rows0 = edge_index[0].astype(jnp.int32)
    cols0 = edge_index[1].astype(jnp.int32)

    lex = jnp.argsort(rows0 * _N_NODES + cols0, stable=True)
    rows = rows0[lex]
    cols = cols0[lex]
    counts = jnp.bincount(rows, length=_N_NODES)
    edge_start = jnp.concatenate(
        [jnp.zeros((1,), dtype=counts.dtype), jnp.cumsum(counts[:-1])]
    )

    logits = x @ w_gate
    top_logits, top_indices = jax.lax.top_k(logits, min(_TOP_K + 1, _NUM_EXPERTS))
    top_k_gates = jax.nn.softmax(top_logits[:, :_TOP_K], axis=1)
    gates = (
        jnp.zeros((_N_NODES, _NUM_EXPERTS), dtype=jnp.float32)
        .at[jnp.arange(_N_NODES)[:, None], top_indices[:, :_TOP_K]]
        .set(top_k_gates)
    )
    importance = gates.sum(0)
    load = (gates > 0).sum(0).astype(jnp.float32)
    loss = _LOSS_COEF * (_cv_squared(importance) + _cv_squared(load))

    W0a = expert_W0[:, :_IN_DIM, :].transpose(1, 0, 2).reshape(_IN_DIM, -1)
    W0b = expert_W0[:, _IN_DIM:2 * _IN_DIM, :].transpose(1, 0, 2).reshape(_IN_DIM, -1)
    W0c = expert_W0[:, 2 * _IN_DIM:, :].transpose(1, 0, 2).reshape(_EDGE_DIM, -1)
    A = x @ W0a
    B = x @ W0b
    hpre = A[rows] + B[cols] + edge_attr[lex] @ W0c + expert_b0.reshape(-1)
    h = jax.nn.relu(hpre).reshape(_N_EDGES, _NUM_EXPERTS, _HIDDEN)
    z = jnp.einsum("neh,eh->ne", h, expert_W1[:, :, 0]) + expert_b1[:, 0]
    z = z / jnp.maximum(jnp.sqrt(jnp.sum(z * z, axis=0)), 1e-12)

    rmax = jax.ops.segment_max(z, rows, num_segments=_N_NODES)
    e = jnp.exp(z - rmax[rows])
    denom = jax.ops.segment_sum(e, rows, num_segments=_N_NODES)
    pi = e / denom[rows]

    k_edges = jnp.asarray(_K_TABLE)[counts]
    edge_end = edge_start[:, None] + k_edges - 1
    rows_b = jnp.broadcast_to(rows[:, None], (_N_EDGES, _NUM_EXPERTS))
    _, _, scores_sorted = jax.lax.sort(
        (rows_b, -pi, pi), dimension=0, num_keys=2, is_stable=False
    )
    thre = jnp.take_along_axis(scores_sorted, edge_end, axis=0)
    mask = (pi - thre[rows] + 1e-15 > 0.0).astype(jnp.float32)

    combined = jnp.sum(gates[rows] * pi * mask, axis=1)
    return combined, loss


# speedup vs baseline: 6.4766x; 1.2234x over previous
import numpy as np
import jax
import jax.numpy as jnp
from jax.experimental import pallas as pl

_N_NODES = 10000
_N_EDGES = 320000
_IN_DIM = 128
_EDGE_DIM = 4
_HIDDEN = 32
_NUM_EXPERTS = 8
_TOP_K = 2
_K_LIST = [0.3, 0.4, 0.5, 0.6, 0.7, 0.8, 0.9, 1.0]
_LOSS_COEF = 0.01

_EH = _NUM_EXPERTS * _HIDDEN
_NPAD = 640

_K_TABLE = np.stack(
    [
        np.maximum(
            np.round(np.arange(_N_EDGES + 1).astype(np.float64) * k).astype(np.int64),
            1,
        )
        for k in _K_LIST
    ],
    axis=1,
).astype(np.int32)


def _cv_squared(v):
    eps = 1e-10
    v = v.astype(jnp.float32)
    return jnp.var(v, ddof=1) / (jnp.mean(v) ** 2 + eps)


_M_BLK = 2000


def _dense_body(x_ref, w_ref, out_ref):
    out_ref[...] = jax.lax.dot_general(
        x_ref[...], w_ref[...], (((1,), (0,)), ((), ())),
        preferred_element_type=jnp.float32,
    )


def _dense_matmul(x, w_pad):
    return pl.pallas_call(
        _dense_body,
        grid=(_N_NODES // _M_BLK,),
        in_specs=[
            pl.BlockSpec((_M_BLK, _IN_DIM), lambda i: (i, 0)),
            pl.BlockSpec((_IN_DIM, _NPAD), lambda i: (0, 0)),
        ],
        out_specs=pl.BlockSpec((_M_BLK, _NPAD), lambda i: (i, 0)),
        out_shape=jax.ShapeDtypeStruct((_N_NODES, _NPAD), jnp.float32),
    )(x, w_pad)


def _gates_body(lg_ref, gates_ref):
    lg = lg_ref[...]
    i1 = jnp.argmax(lg, axis=1)
    m1 = jnp.max(lg, axis=1)
    eidx = jax.lax.broadcasted_iota(jnp.int32, lg.shape, 1)
    lg2 = jnp.where(eidx == i1[:, None], -jnp.inf, lg)
    i2 = jnp.argmax(lg2, axis=1)
    m2 = jnp.max(lg2, axis=1)
    e2 = jnp.exp(m2 - m1)
    g1 = 1.0 / (1.0 + e2)
    g2 = e2 / (1.0 + e2)
    gates_ref[...] = (
        jnp.where(eidx == i1[:, None], g1[:, None], 0.0)
        + jnp.where(eidx == i2[:, None], g2[:, None], 0.0)
    ).astype(jnp.float32)


def _gates_kernel(logits):
    return pl.pallas_call(
        _gates_body,
        grid=(_N_NODES // _M_BLK,),
        in_specs=[pl.BlockSpec((_M_BLK, _NUM_EXPERTS), lambda i: (i, 0))],
        out_specs=pl.BlockSpec((_M_BLK, _NUM_EXPERTS), lambda i: (i, 0)),
        out_shape=jax.ShapeDtypeStruct((_N_NODES, _NUM_EXPERTS), jnp.float32),
    )(logits)


_E_BLK = 8000


def _zmlp_body(hs_ref, ea_ref, w0c_ref, b0_ref, w1_ref, b1_ref, z_ref, zsq_ref):
    h = hs_ref[...] + jax.lax.dot_general(
        ea_ref[...], w0c_ref[...], (((1,), (0,)), ((), ())),
        preferred_element_type=jnp.float32,
    ) + b0_ref[...]
    h = jnp.maximum(h, 0.0)
    z = jax.lax.dot_general(
        h, w1_ref[...], (((1,), (0,)), ((), ())),
        preferred_element_type=jnp.float32,
    ) + b1_ref[...]
    z_ref[...] = z

    @pl.when(pl.program_id(0) == 0)
    def _init():
        zsq_ref[...] = jnp.zeros_like(zsq_ref)

    zsq_ref[...] += jnp.sum(z * z, axis=0, keepdims=True)


def _zmlp_kernel(hsum, ea_lex, w0c, b0, w1bd, b1):
    return pl.pallas_call(
        _zmlp_body,
        grid=(_N_EDGES // _E_BLK,),
        in_specs=[
            pl.BlockSpec((_E_BLK, _EH), lambda i: (i, 0)),
            pl.BlockSpec((_E_BLK, _EDGE_DIM), lambda i: (i, 0)),
            pl.BlockSpec((_EDGE_DIM, _EH), lambda i: (0, 0)),
            pl.BlockSpec((1, _EH), lambda i: (0, 0)),
            pl.BlockSpec((_EH, _NUM_EXPERTS), lambda i: (0, 0)),
            pl.BlockSpec((1, _NUM_EXPERTS), lambda i: (0, 0)),
        ],
        out_specs=[
            pl.BlockSpec((_E_BLK, _NUM_EXPERTS), lambda i: (i, 0)),
            pl.BlockSpec((1, _NUM_EXPERTS), lambda i: (0, 0)),
        ],
        out_shape=[
            jax.ShapeDtypeStruct((_N_EDGES, _NUM_EXPERTS), jnp.float32),
            jax.ShapeDtypeStruct((1, _NUM_EXPERTS), jnp.float32),
        ],
    )(hsum, ea_lex, w0c, b0, w1bd, b1)


def _combine_body(pi_ref, th_ref, g_ref, out_ref):
    pi = pi_ref[...]
    mask = (pi - th_ref[...] + 1e-15 > 0.0).astype(jnp.float32)
    out_ref[...] = jnp.sum(g_ref[...] * pi * mask, axis=0)


_C_ROWS = _N_EDGES // _E_BLK
_C_BLK = 8


def _combine_kernel(pi_t, thre_lex_t, gates_lex_t):
    r3 = lambda a: a.reshape(_NUM_EXPERTS, _C_ROWS, _E_BLK)
    spec = pl.BlockSpec((_NUM_EXPERTS, _C_BLK, _E_BLK), lambda i: (0, i, 0))
    return pl.pallas_call(
        _combine_body,
        grid=(_C_ROWS // _C_BLK,),
        in_specs=[spec, spec, spec],
        out_specs=pl.BlockSpec((_C_BLK, _E_BLK), lambda i: (i, 0)),
        out_shape=jax.ShapeDtypeStruct((_C_ROWS, _E_BLK), jnp.float32),
    )(r3(pi_t), r3(thre_lex_t), r3(gates_lex_t)).reshape(_N_EDGES)


def kernel(x, edge_index, edge_attr, w_gate, w_noise, expert_W0, expert_b0,
           expert_W1, expert_b1):
    rows0 = edge_index[0].astype(jnp.int32)
    cols0 = edge_index[1].astype(jnp.int32)

    lex = jnp.argsort(rows0 * _N_NODES + cols0, stable=True)
    rows = rows0[lex]
    cols = cols0[lex]
    counts = jnp.bincount(rows, length=_N_NODES)
    edge_start = jnp.concatenate(
        [jnp.zeros((1,), dtype=counts.dtype), jnp.cumsum(counts[:-1])]
    )

    W0a = expert_W0[:, :_IN_DIM, :].transpose(1, 0, 2).reshape(_IN_DIM, _EH)
    W0b = expert_W0[:, _IN_DIM:2 * _IN_DIM, :].transpose(1, 0, 2).reshape(_IN_DIM, _EH)
    w_pad = jnp.concatenate(
        [W0a, W0b, w_gate,
         jnp.zeros((_IN_DIM, _NPAD - 2 * _EH - _NUM_EXPERTS), jnp.float32)],
        axis=1,
    )
    dense = _dense_matmul(x, w_pad)
    A = dense[:, :_EH]
    B = dense[:, _EH:2 * _EH]
    logits = dense[:, 2 * _EH:2 * _EH + _NUM_EXPERTS]

    gates = _gates_kernel(logits)
    importance = gates.sum(0)
    load = (gates > 0).sum(0).astype(jnp.float32)
    loss = _LOSS_COEF * (_cv_squared(importance) + _cv_squared(load))

    W0c = expert_W0[:, 2 * _IN_DIM:, :].transpose(1, 0, 2).reshape(_EDGE_DIM, _EH)
    b0 = expert_b0.reshape(1, _EH)
    w1bd = jnp.zeros((_EH, _NUM_EXPERTS), jnp.float32)
    eidx = jnp.repeat(jnp.arange(_NUM_EXPERTS), _HIDDEN)
    w1bd = w1bd.at[jnp.arange(_EH), eidx].set(expert_W1[:, :, 0].reshape(_EH))
    b1 = expert_b1[:, 0].reshape(1, _NUM_EXPERTS)

    hsum = A[rows] + B[cols]
    z, zsq = _zmlp_kernel(hsum, edge_attr[lex], W0c, b0, w1bd, b1)
    z = z / jnp.maximum(jnp.sqrt(zsq[0]), 1e-12)

    rmax = jax.ops.segment_max(z, rows, num_segments=_N_NODES)
    e = jnp.exp(z - rmax[rows])
    denom = jax.ops.segment_sum(e, rows, num_segments=_N_NODES)
    pi = e / denom[rows]

    k_edges = jnp.asarray(_K_TABLE)[counts]
    edge_end = edge_start[:, None] + k_edges - 1
    rows_b = jnp.broadcast_to(rows[:, None], (_N_EDGES, _NUM_EXPERTS))
    _, _, scores_sorted = jax.lax.sort(
        (rows_b, -pi, pi), dimension=0, num_keys=2, is_stable=False
    )
    thre = jnp.take_along_axis(scores_sorted, edge_end, axis=0)

    combined = _combine_kernel(pi.T, thre[rows].T, gates[rows].T)
    return combined, loss


# speedup vs baseline: 6.5531x; 1.0118x over previous
import functools

import numpy as np
import jax
import jax.numpy as jnp
from jax import lax
from jax.experimental import pallas as pl
from jax.experimental.pallas import tpu as pltpu
from jax.experimental.pallas import tpu_sc as plsc

_N_NODES = 10000
_N_EDGES = 320000
_IN_DIM = 128
_EDGE_DIM = 4
_HIDDEN = 32
_NUM_EXPERTS = 8
_TOP_K = 2
_K_LIST = [0.3, 0.4, 0.5, 0.6, 0.7, 0.8, 0.9, 1.0]
_LOSS_COEF = 0.01

_EH = _NUM_EXPERTS * _HIDDEN
_NPAD = 640

_K_TABLE = np.stack(
    [
        np.maximum(
            np.round(np.arange(_N_EDGES + 1).astype(np.float64) * k).astype(np.int64),
            1,
        )
        for k in _K_LIST
    ],
    axis=1,
).astype(np.int32)


def _cv_squared(v):
    eps = 1e-10
    v = v.astype(jnp.float32)
    return jnp.var(v, ddof=1) / (jnp.mean(v) ** 2 + eps)


_M_BLK = 2000


def _dense_body(x_ref, w_ref, out_ref):
    out_ref[...] = jax.lax.dot_general(
        x_ref[...], w_ref[...], (((1,), (0,)), ((), ())),
        preferred_element_type=jnp.float32,
    )


def _dense_matmul(x, w_pad):
    return pl.pallas_call(
        _dense_body,
        grid=(_N_NODES // _M_BLK,),
        in_specs=[
            pl.BlockSpec((_M_BLK, _IN_DIM), lambda i: (i, 0)),
            pl.BlockSpec((_IN_DIM, _NPAD), lambda i: (0, 0)),
        ],
        out_specs=pl.BlockSpec((_M_BLK, _NPAD), lambda i: (i, 0)),
        out_shape=jax.ShapeDtypeStruct((_N_NODES, _NPAD), jnp.float32),
    )(x, w_pad)


def _gates_body(lg_ref, gates_ref):
    lg = lg_ref[...]
    i1 = jnp.argmax(lg, axis=1)
    m1 = jnp.max(lg, axis=1)
    eidx = jax.lax.broadcasted_iota(jnp.int32, lg.shape, 1)
    lg2 = jnp.where(eidx == i1[:, None], -jnp.inf, lg)
    i2 = jnp.argmax(lg2, axis=1)
    m2 = jnp.max(lg2, axis=1)
    e2 = jnp.exp(m2 - m1)
    g1 = 1.0 / (1.0 + e2)
    g2 = e2 / (1.0 + e2)
    gates_ref[...] = (
        jnp.where(eidx == i1[:, None], g1[:, None], 0.0)
        + jnp.where(eidx == i2[:, None], g2[:, None], 0.0)
    ).astype(jnp.float32)


def _gates_kernel(logits):
    return pl.pallas_call(
        _gates_body,
        grid=(_N_NODES // _M_BLK,),
        in_specs=[pl.BlockSpec((_M_BLK, _NUM_EXPERTS), lambda i: (i, 0))],
        out_specs=pl.BlockSpec((_M_BLK, _NUM_EXPERTS), lambda i: (i, 0)),
        out_shape=jax.ShapeDtypeStruct((_N_NODES, _NUM_EXPERTS), jnp.float32),
    )(logits)


_E_BLK = 8000


def _zmlp_body(hs_ref, ea_ref, w0c_ref, b0_ref, w1_ref, b1_ref, z_ref, zsq_ref):
    h = hs_ref[...] + jax.lax.dot_general(
        ea_ref[...], w0c_ref[...], (((1,), (0,)), ((), ())),
        preferred_element_type=jnp.float32,
    ) + b0_ref[...]
    h = jnp.maximum(h, 0.0)
    z = jax.lax.dot_general(
        h, w1_ref[...], (((1,), (0,)), ((), ())),
        preferred_element_type=jnp.float32,
    ) + b1_ref[...]
    z_ref[...] = z

    @pl.when(pl.program_id(0) == 0)
    def _init():
        zsq_ref[...] = jnp.zeros_like(zsq_ref)

    zsq_ref[...] += jnp.sum(z * z, axis=0, keepdims=True)


def _zmlp_kernel(hsum, ea_lex, w0c, b0, w1bd, b1):
    return pl.pallas_call(
        _zmlp_body,
        grid=(_N_EDGES // _E_BLK,),
        in_specs=[
            pl.BlockSpec((_E_BLK, _EH), lambda i: (i, 0)),
            pl.BlockSpec((_E_BLK, _EDGE_DIM), lambda i: (i, 0)),
            pl.BlockSpec((_EDGE_DIM, _EH), lambda i: (0, 0)),
            pl.BlockSpec((1, _EH), lambda i: (0, 0)),
            pl.BlockSpec((_EH, _NUM_EXPERTS), lambda i: (0, 0)),
            pl.BlockSpec((1, _NUM_EXPERTS), lambda i: (0, 0)),
        ],
        out_specs=[
            pl.BlockSpec((_E_BLK, _NUM_EXPERTS), lambda i: (i, 0)),
            pl.BlockSpec((1, _NUM_EXPERTS), lambda i: (0, 0)),
        ],
        out_shape=[
            jax.ShapeDtypeStruct((_N_EDGES, _NUM_EXPERTS), jnp.float32),
            jax.ShapeDtypeStruct((1, _NUM_EXPERTS), jnp.float32),
        ],
    )(hsum, ea_lex, w0c, b0, w1bd, b1)


def _combine_body(pi_ref, th_ref, g_ref, out_ref):
    pi = pi_ref[...]
    mask = (pi - th_ref[...] + 1e-15 > 0.0).astype(jnp.float32)
    out_ref[...] = jnp.sum(g_ref[...] * pi * mask, axis=0)


_C_ROWS = _N_EDGES // _E_BLK
_C_BLK = 8


def _combine_kernel(pi_t, thre_lex_t, gates_lex_t):
    r3 = lambda a: a.reshape(_NUM_EXPERTS, _C_ROWS, _E_BLK)
    spec = pl.BlockSpec((_NUM_EXPERTS, _C_BLK, _E_BLK), lambda i: (0, i, 0))
    return pl.pallas_call(
        _combine_body,
        grid=(_C_ROWS // _C_BLK,),
        in_specs=[spec, spec, spec],
        out_specs=pl.BlockSpec((_C_BLK, _E_BLK), lambda i: (i, 0)),
        out_shape=jax.ShapeDtypeStruct((_C_ROWS, _E_BLK), jnp.float32),
    )(r3(pi_t), r3(thre_lex_t), r3(gates_lex_t)).reshape(_N_EDGES)


_N_WORKERS = 32
_R_PER_W = 313
_RW8 = 2512
_CH = 256
_CH8 = _CH * 8
_E_PAD = 512
_LO0 = -1065353217
_HI0 = 1065353217

_sc_mesh = plsc.VectorSubcoreMesh(core_axis_name="c", subcore_axis_name="s")


@functools.partial(
    pl.kernel, mesh=_sc_mesh,
    compiler_params=pltpu.CompilerParams(needs_layout_passes=False),
    out_type=jax.ShapeDtypeStruct((_N_WORKERS, _RW8), jnp.int32),
    scratch_types=[
        pltpu.VMEM((_CH8,), jnp.int32),
        pltpu.VMEM((_CH8,), jnp.int32),
        pltpu.VMEM((_RW8,), jnp.int32),
        pltpu.VMEM((_RW8,), jnp.int32),
        pltpu.VMEM((_RW8,), jnp.int32),
        pltpu.VMEM((_RW8,), jnp.int32),
        pltpu.VMEM((_RW8,), jnp.int32),
        pltpu.VMEM((32,), jnp.int32),
        pltpu.VMEM((32,), jnp.int32),
    ],
)
def _select_kernel(zi_hbm, idx_hbm, k_hbm, astart_hbm, wcnt_hbm, out_hbm,
                   zi_v, idx_v, lo_v, hi_v, mid_v, cnt_v, k_v, as_v, wc_v):
    w = lax.axis_index("s") * 2 + lax.axis_index("c")
    pltpu.sync_copy(k_hbm.at[pl.ds(w * _RW8, _RW8)], k_v)
    pltpu.sync_copy(astart_hbm, as_v)
    pltpu.sync_copy(wcnt_hbm, wc_v)

    lanes = lax.iota(jnp.int32, 16)

    def _scal(ref):
        a = jnp.sum(jnp.where(lanes == w, ref[pl.ds(0, 16)], 0))
        b = jnp.sum(jnp.where(lanes + 16 == w, ref[pl.ds(16, 16)], 0))
        return a + b

    astart = _scal(as_v)
    nch = _scal(wc_v)
    r0x8 = w * (_R_PER_W * 8)

    nv = _RW8 // 16

    def _fill(i, _):
        lo_v[pl.ds(i * 16, 16)] = jnp.full((16,), _LO0, jnp.int32)
        hi_v[pl.ds(i * 16, 16)] = jnp.full((16,), _HI0, jnp.int32)
        return 0

    lax.fori_loop(0, nv, _fill, 0)

    ones = jnp.ones((16,), jnp.int32)

    def _iter(it, _):
        def _prep(i, _):
            s = pl.ds(i * 16, 16)
            lo = lo_v[s]
            mid_v[s] = lo + lax.shift_right_arithmetic(hi_v[s] - lo, 1)
            cnt_v[s] = jnp.zeros((16,), jnp.int32)
            return 0

        lax.fori_loop(0, nv, _prep, 0)

        def _cond(carry):
            return carry[0] < nch

        def _chunk(carry):
            c, base = carry
            pltpu.sync_copy(zi_hbm.at[pl.ds(base * 8, _CH8)], zi_v)
            pltpu.sync_copy(idx_hbm.at[pl.ds(base * 8, _CH8)], idx_v)

            def _sweep(i, _):
                s = pl.ds(i * 16, 16)
                zi = zi_v[s]
                il = idx_v[s] - r0x8
                valid = (il >= 0) & (il < _R_PER_W * 8)
                midg = plsc.load_gather(mid_v, [il], mask=valid)
                pred = valid & (zi >= midg)
                plsc.addupdate_scatter(cnt_v, [il], ones, mask=pred)
                return 0

            lax.fori_loop(0, _CH8 // 16, _sweep, 0)
            return (c + 1, base + _CH)

        lax.while_loop(_cond, _chunk, (0, astart))

        def _upd(i, _):
            s = pl.ds(i * 16, 16)
            ge = cnt_v[s] >= k_v[s]
            m = mid_v[s]
            lo_v[s] = jnp.where(ge, m, lo_v[s])
            hi_v[s] = jnp.where(ge, hi_v[s], m)
            return 0

        lax.fori_loop(0, nv, _upd, 0)
        return 0

    lax.fori_loop(0, 31, _iter, 0)

    pltpu.sync_copy(lo_v, out_hbm.at[w])


def _sortable_keys(z):
    b = lax.bitcast_convert_type(z, jnp.int32)
    return jnp.where(b >= 0, b, b ^ jnp.int32(0x7FFFFFFF))


def _sc_thresholds(zi, rows, edge_start, k_edges):
    pad_e = jnp.full((_E_PAD, _NUM_EXPERTS), _LO0, jnp.int32)
    zi8 = jnp.concatenate([zi, pad_e], axis=0).reshape(-1)
    idx8 = rows[:, None] * 8 + jnp.arange(8, dtype=jnp.int32)[None, :]
    pad_i = (10015 * 8 + jnp.arange(8, dtype=jnp.int32))[None, :] * jnp.ones(
        (_E_PAD, 1), jnp.int32)
    idx8 = jnp.concatenate([idx8, pad_i], axis=0).reshape(-1)

    kp = jnp.concatenate(
        [k_edges, jnp.ones((_N_WORKERS * _R_PER_W - _N_NODES, _NUM_EXPERTS),
                           jnp.int32)], axis=0)
    kp = kp.reshape(_N_WORKERS, _R_PER_W * 8)
    kp = jnp.concatenate(
        [kp, jnp.ones((_N_WORKERS, _RW8 - _R_PER_W * 8), jnp.int32)], axis=1
    ).reshape(-1)

    bnd = jnp.concatenate(
        [edge_start[jnp.arange(_N_WORKERS) * _R_PER_W],
         jnp.array([_N_EDGES], jnp.int32)])
    astart = (bnd[:_N_WORKERS] // 8) * 8
    wcnt = (bnd[1:] - astart + _CH - 1) // _CH

    out = _select_kernel(zi8, idx8, kp,
                         astart.astype(jnp.int32), wcnt.astype(jnp.int32))
    thre = out[:, :_R_PER_W * 8].reshape(-1, _NUM_EXPERTS)[:_N_NODES]
    return thre


def kernel(x, edge_index, edge_attr, w_gate, w_noise, expert_W0, expert_b0,
           expert_W1, expert_b1):
    rows0 = edge_index[0].astype(jnp.int32)
    cols0 = edge_index[1].astype(jnp.int32)

    lex = jnp.argsort(rows0 * _N_NODES + cols0, stable=True)
    rows = rows0[lex]
    cols = cols0[lex]
    counts = jnp.bincount(rows, length=_N_NODES)
    edge_start = jnp.concatenate(
        [jnp.zeros((1,), dtype=counts.dtype), jnp.cumsum(counts[:-1])]
    )

    W0a = expert_W0[:, :_IN_DIM, :].transpose(1, 0, 2).reshape(_IN_DIM, _EH)
    W0b = expert_W0[:, _IN_DIM:2 * _IN_DIM, :].transpose(1, 0, 2).reshape(_IN_DIM, _EH)
    w_pad = jnp.concatenate(
        [W0a, W0b, w_gate,
         jnp.zeros((_IN_DIM, _NPAD - 2 * _EH - _NUM_EXPERTS), jnp.float32)],
        axis=1,
    )
    dense = _dense_matmul(x, w_pad)
    A = dense[:, :_EH]
    B = dense[:, _EH:2 * _EH]
    logits = dense[:, 2 * _EH:2 * _EH + _NUM_EXPERTS]

    gates = _gates_kernel(logits)
    importance = gates.sum(0)
    load = (gates > 0).sum(0).astype(jnp.float32)
    loss = _LOSS_COEF * (_cv_squared(importance) + _cv_squared(load))

    W0c = expert_W0[:, 2 * _IN_DIM:, :].transpose(1, 0, 2).reshape(_EDGE_DIM, _EH)
    b0 = expert_b0.reshape(1, _EH)
    w1bd = jnp.zeros((_EH, _NUM_EXPERTS), jnp.float32)
    eidx = jnp.repeat(jnp.arange(_NUM_EXPERTS), _HIDDEN)
    w1bd = w1bd.at[jnp.arange(_EH), eidx].set(expert_W1[:, :, 0].reshape(_EH))
    b1 = expert_b1[:, 0].reshape(1, _NUM_EXPERTS)

    hsum = A[rows] + B[cols]
    z, zsq = _zmlp_kernel(hsum, edge_attr[lex], W0c, b0, w1bd, b1)
    z = z / jnp.maximum(jnp.sqrt(zsq[0]), 1e-12)

    rmax = jax.ops.segment_max(z, rows, num_segments=_N_NODES)
    e = jnp.exp(z - rmax[rows])
    denom = jax.ops.segment_sum(e, rows, num_segments=_N_NODES)
    pi = e / denom[rows]

    k_edges = jnp.asarray(_K_TABLE)[counts]
    zi = _sortable_keys(z)
    thre_key = _sc_thresholds(zi, rows, edge_start.astype(jnp.int32), k_edges)
    vk = lax.bitcast_convert_type(
        jnp.where(thre_key >= 0, thre_key, thre_key ^ jnp.int32(0x7FFFFFFF)),
        jnp.float32)
    thre = jnp.exp(vk - rmax) / denom

    combined = _combine_kernel(pi.T, thre[rows].T, gates[rows].T)
    return combined, loss


# speedup vs baseline: 7.2526x; 1.1067x over previous
import functools

import numpy as np
import jax
import jax.numpy as jnp
from jax import lax
from jax.experimental import pallas as pl
from jax.experimental.pallas import tpu as pltpu
from jax.experimental.pallas import tpu_sc as plsc

_N_NODES = 10000
_N_EDGES = 320000
_IN_DIM = 128
_EDGE_DIM = 4
_HIDDEN = 32
_NUM_EXPERTS = 8
_TOP_K = 2
_K_LIST = [0.3, 0.4, 0.5, 0.6, 0.7, 0.8, 0.9, 1.0]
_LOSS_COEF = 0.01

_EH = _NUM_EXPERTS * _HIDDEN
_NPAD = 640

_K_TABLE = np.stack(
    [
        np.maximum(
            np.round(np.arange(_N_EDGES + 1).astype(np.float64) * k).astype(np.int64),
            1,
        )
        for k in _K_LIST
    ],
    axis=1,
).astype(np.int32)


def _cv_squared(v):
    eps = 1e-10
    v = v.astype(jnp.float32)
    return jnp.var(v, ddof=1) / (jnp.mean(v) ** 2 + eps)


_M_BLK = 2000


def _dense_body(x_ref, w_ref, out_ref):
    out_ref[...] = jax.lax.dot_general(
        x_ref[...], w_ref[...], (((1,), (0,)), ((), ())),
        preferred_element_type=jnp.float32,
    )


def _dense_matmul(x, w_pad):
    return pl.pallas_call(
        _dense_body,
        grid=(_N_NODES // _M_BLK,),
        in_specs=[
            pl.BlockSpec((_M_BLK, _IN_DIM), lambda i: (i, 0)),
            pl.BlockSpec((_IN_DIM, _NPAD), lambda i: (0, 0)),
        ],
        out_specs=pl.BlockSpec((_M_BLK, _NPAD), lambda i: (i, 0)),
        out_shape=jax.ShapeDtypeStruct((_N_NODES, _NPAD), jnp.float32),
    )(x, w_pad)


def _gates_body(lg_ref, gates_ref):
    lg = lg_ref[...]
    i1 = jnp.argmax(lg, axis=1)
    m1 = jnp.max(lg, axis=1)
    eidx = jax.lax.broadcasted_iota(jnp.int32, lg.shape, 1)
    lg2 = jnp.where(eidx == i1[:, None], -jnp.inf, lg)
    i2 = jnp.argmax(lg2, axis=1)
    m2 = jnp.max(lg2, axis=1)
    e2 = jnp.exp(m2 - m1)
    g1 = 1.0 / (1.0 + e2)
    g2 = e2 / (1.0 + e2)
    gates_ref[...] = (
        jnp.where(eidx == i1[:, None], g1[:, None], 0.0)
        + jnp.where(eidx == i2[:, None], g2[:, None], 0.0)
    ).astype(jnp.float32)


def _gates_kernel(logits):
    return pl.pallas_call(
        _gates_body,
        grid=(_N_NODES // _M_BLK,),
        in_specs=[pl.BlockSpec((_M_BLK, _NUM_EXPERTS), lambda i: (i, 0))],
        out_specs=pl.BlockSpec((_M_BLK, _NUM_EXPERTS), lambda i: (i, 0)),
        out_shape=jax.ShapeDtypeStruct((_N_NODES, _NUM_EXPERTS), jnp.float32),
    )(logits)


_E_BLK = 8000


def _zmlp_body(hs_ref, ea_ref, w0c_ref, b0_ref, w1_ref, b1_ref, z_ref, zsq_ref):
    h = hs_ref[...] + jax.lax.dot_general(
        ea_ref[...], w0c_ref[...], (((1,), (0,)), ((), ())),
        preferred_element_type=jnp.float32,
    ) + b0_ref[...]
    h = jnp.maximum(h, 0.0)
    z = jax.lax.dot_general(
        h, w1_ref[...], (((1,), (0,)), ((), ())),
        preferred_element_type=jnp.float32,
    ) + b1_ref[...]
    z_ref[...] = z

    @pl.when(pl.program_id(0) == 0)
    def _init():
        zsq_ref[...] = jnp.zeros_like(zsq_ref)

    zsq_ref[...] += jnp.sum(z * z, axis=0, keepdims=True)


def _zmlp_kernel(hsum, ea_lex, w0c, b0, w1bd, b1):
    return pl.pallas_call(
        _zmlp_body,
        grid=(_N_EDGES // _E_BLK,),
        in_specs=[
            pl.BlockSpec((_E_BLK, _EH), lambda i: (i, 0)),
            pl.BlockSpec((_E_BLK, _EDGE_DIM), lambda i: (i, 0)),
            pl.BlockSpec((_EDGE_DIM, _EH), lambda i: (0, 0)),
            pl.BlockSpec((1, _EH), lambda i: (0, 0)),
            pl.BlockSpec((_EH, _NUM_EXPERTS), lambda i: (0, 0)),
            pl.BlockSpec((1, _NUM_EXPERTS), lambda i: (0, 0)),
        ],
        out_specs=[
            pl.BlockSpec((_E_BLK, _NUM_EXPERTS), lambda i: (i, 0)),
            pl.BlockSpec((1, _NUM_EXPERTS), lambda i: (0, 0)),
        ],
        out_shape=[
            jax.ShapeDtypeStruct((_N_EDGES, _NUM_EXPERTS), jnp.float32),
            jax.ShapeDtypeStruct((1, _NUM_EXPERTS), jnp.float32),
        ],
    )(hsum, ea_lex, w0c, b0, w1bd, b1)


def _combine_body(pi_ref, th_ref, g_ref, out_ref):
    pi = pi_ref[...]
    mask = (pi - th_ref[...] + 1e-15 > 0.0).astype(jnp.float32)
    out_ref[...] = jnp.sum(g_ref[...] * pi * mask, axis=0)


_C_ROWS = _N_EDGES // _E_BLK
_C_BLK = 8


def _combine_kernel(pi_t, thre_lex_t, gates_lex_t):
    r3 = lambda a: a.reshape(_NUM_EXPERTS, _C_ROWS, _E_BLK)
    spec = pl.BlockSpec((_NUM_EXPERTS, _C_BLK, _E_BLK), lambda i: (0, i, 0))
    return pl.pallas_call(
        _combine_body,
        grid=(_C_ROWS // _C_BLK,),
        in_specs=[spec, spec, spec],
        out_specs=pl.BlockSpec((_C_BLK, _E_BLK), lambda i: (i, 0)),
        out_shape=jax.ShapeDtypeStruct((_C_ROWS, _E_BLK), jnp.float32),
    )(r3(pi_t), r3(thre_lex_t), r3(gates_lex_t)).reshape(_N_EDGES)


_N_WORKERS = 32
_R_PER_W = 313
_RW8 = 2512
_CH = 1024
_CH8 = _CH * 8
_E_PAD = 2048
_LO0 = -1065353217
_HI0 = 1065353217

@functools.cache
def _get_select_kernel():
    mesh = plsc.VectorSubcoreMesh(core_axis_name="c", subcore_axis_name="s")
    return functools.partial(
        pl.kernel, mesh=mesh,
        compiler_params=pltpu.CompilerParams(needs_layout_passes=False),
        out_type=jax.ShapeDtypeStruct((_N_WORKERS, _RW8), jnp.int32),
        scratch_types=[
            pltpu.VMEM((_CH8,), jnp.int32),
            pltpu.VMEM((_CH8,), jnp.int32),
            pltpu.VMEM((_RW8,), jnp.int32),
            pltpu.VMEM((_RW8,), jnp.int32),
            pltpu.VMEM((_RW8,), jnp.int32),
            pltpu.VMEM((_RW8,), jnp.int32),
            pltpu.VMEM((_RW8,), jnp.int32),
            pltpu.VMEM((32,), jnp.int32),
            pltpu.VMEM((32,), jnp.int32),
        ],
    )(_select_body)


def _select_body(zi_hbm, idx_hbm, k_hbm, astart_hbm, wcnt_hbm, out_hbm,
                   zi_v, idx_v, lo_v, hi_v, mid_v, cnt_v, k_v, as_v, wc_v):
    w = lax.axis_index("s") * 2 + lax.axis_index("c")
    pltpu.sync_copy(k_hbm.at[pl.ds(w * _RW8, _RW8)], k_v)
    pltpu.sync_copy(astart_hbm, as_v)
    pltpu.sync_copy(wcnt_hbm, wc_v)

    lanes = lax.iota(jnp.int32, 16)

    def _scal(ref):
        a = jnp.sum(jnp.where(lanes == w, ref[pl.ds(0, 16)], 0))
        b = jnp.sum(jnp.where(lanes + 16 == w, ref[pl.ds(16, 16)], 0))
        return a + b

    astart = _scal(as_v)
    nch = _scal(wc_v)
    r0x8 = w * (_R_PER_W * 8)

    nv = _RW8 // 16

    def _fill(i, _):
        lo_v[pl.ds(i * 16, 16)] = jnp.full((16,), _LO0, jnp.int32)
        hi_v[pl.ds(i * 16, 16)] = jnp.full((16,), _HI0, jnp.int32)
        return 0

    lax.fori_loop(0, nv, _fill, 0)

    ones = jnp.ones((16,), jnp.int32)

    def _iter(it, _):
        def _prep(i, _):
            s = pl.ds(i * 16, 16)
            lo = lo_v[s]
            mid_v[s] = lo + lax.shift_right_arithmetic(hi_v[s] - lo, 1)
            cnt_v[s] = jnp.zeros((16,), jnp.int32)
            return 0

        lax.fori_loop(0, nv, _prep, 0)

        def _cond(carry):
            return carry[0] < nch

        def _chunk(carry):
            c, base = carry
            pltpu.sync_copy(zi_hbm.at[pl.ds(base * 8, _CH8)], zi_v)
            pltpu.sync_copy(idx_hbm.at[pl.ds(base * 8, _CH8)], idx_v)

            def _sweep(i, _):
                for u in range(4):
                    s = pl.ds(i * 64 + u * 16, 16)
                    zi = zi_v[s]
                    il = idx_v[s] - r0x8
                    valid = (il >= 0) & (il < _R_PER_W * 8)
                    midg = plsc.load_gather(mid_v, [il], mask=valid)
                    pred = valid & (zi >= midg)
                    plsc.addupdate_scatter(cnt_v, [il], ones, mask=pred)
                return 0

            lax.fori_loop(0, _CH8 // 64, _sweep, 0)
            return (c + 1, base + _CH)

        lax.while_loop(_cond, _chunk, (0, astart))

        def _upd(i, _):
            s = pl.ds(i * 16, 16)
            ge = cnt_v[s] >= k_v[s]
            m = mid_v[s]
            lo_v[s] = jnp.where(ge, m, lo_v[s])
            hi_v[s] = jnp.where(ge, hi_v[s], m)
            return 0

        lax.fori_loop(0, nv, _upd, 0)
        return 0

    lax.fori_loop(0, 31, _iter, 0)

    pltpu.sync_copy(lo_v, out_hbm.at[w])


def _sortable_keys(z):
    b = lax.bitcast_convert_type(z, jnp.int32)
    return jnp.where(b >= 0, b, b ^ jnp.int32(0x7FFFFFFF))


def _sc_thresholds(zi, rows, edge_start, k_edges):
    pad_e = jnp.full((_E_PAD, _NUM_EXPERTS), _LO0, jnp.int32)
    zi8 = jnp.concatenate([zi, pad_e], axis=0).reshape(-1)
    idx8 = rows[:, None] * 8 + jnp.arange(8, dtype=jnp.int32)[None, :]
    pad_i = (10015 * 8 + jnp.arange(8, dtype=jnp.int32))[None, :] * jnp.ones(
        (_E_PAD, 1), jnp.int32)
    idx8 = jnp.concatenate([idx8, pad_i], axis=0).reshape(-1)

    kp = jnp.concatenate(
        [k_edges, jnp.ones((_N_WORKERS * _R_PER_W - _N_NODES, _NUM_EXPERTS),
                           jnp.int32)], axis=0)
    kp = kp.reshape(_N_WORKERS, _R_PER_W * 8)
    kp = jnp.concatenate(
        [kp, jnp.ones((_N_WORKERS, _RW8 - _R_PER_W * 8), jnp.int32)], axis=1
    ).reshape(-1)

    bnd = jnp.concatenate(
        [edge_start[jnp.arange(_N_WORKERS) * _R_PER_W],
         jnp.array([_N_EDGES], jnp.int32)])
    astart = (bnd[:_N_WORKERS] // 8) * 8
    wcnt = (bnd[1:] - astart + _CH - 1) // _CH

    out = _get_select_kernel()(zi8, idx8, kp,
                               astart.astype(jnp.int32), wcnt.astype(jnp.int32))
    thre = out[:, :_R_PER_W * 8].reshape(-1, _NUM_EXPERTS)[:_N_NODES]
    return thre


def kernel(x, edge_index, edge_attr, w_gate, w_noise, expert_W0, expert_b0,
           expert_W1, expert_b1):
    rows0 = edge_index[0].astype(jnp.int32)
    cols0 = edge_index[1].astype(jnp.int32)

    lex = jnp.argsort(rows0 * _N_NODES + cols0, stable=True)
    rows = rows0[lex]
    cols = cols0[lex]
    counts = jnp.bincount(rows, length=_N_NODES)
    edge_start = jnp.concatenate(
        [jnp.zeros((1,), dtype=counts.dtype), jnp.cumsum(counts[:-1])]
    )

    W0a = expert_W0[:, :_IN_DIM, :].transpose(1, 0, 2).reshape(_IN_DIM, _EH)
    W0b = expert_W0[:, _IN_DIM:2 * _IN_DIM, :].transpose(1, 0, 2).reshape(_IN_DIM, _EH)
    w_pad = jnp.concatenate(
        [W0a, W0b, w_gate,
         jnp.zeros((_IN_DIM, _NPAD - 2 * _EH - _NUM_EXPERTS), jnp.float32)],
        axis=1,
    )
    dense = _dense_matmul(x, w_pad)
    A = dense[:, :_EH]
    B = dense[:, _EH:2 * _EH]
    logits = dense[:, 2 * _EH:2 * _EH + _NUM_EXPERTS]

    gates = _gates_kernel(logits)
    importance = gates.sum(0)
    load = (gates > 0).sum(0).astype(jnp.float32)
    loss = _LOSS_COEF * (_cv_squared(importance) + _cv_squared(load))

    W0c = expert_W0[:, 2 * _IN_DIM:, :].transpose(1, 0, 2).reshape(_EDGE_DIM, _EH)
    b0 = expert_b0.reshape(1, _EH)
    w1bd = jnp.zeros((_EH, _NUM_EXPERTS), jnp.float32)
    eidx = jnp.repeat(jnp.arange(_NUM_EXPERTS), _HIDDEN)
    w1bd = w1bd.at[jnp.arange(_EH), eidx].set(expert_W1[:, :, 0].reshape(_EH))
    b1 = expert_b1[:, 0].reshape(1, _NUM_EXPERTS)

    hsum = A[rows] + B[cols]
    z, zsq = _zmlp_kernel(hsum, edge_attr[lex], W0c, b0, w1bd, b1)
    z = z / jnp.maximum(jnp.sqrt(zsq[0]), 1e-12)

    rmax = jax.ops.segment_max(z, rows, num_segments=_N_NODES)
    e = jnp.exp(z - rmax[rows])
    denom = jax.ops.segment_sum(e, rows, num_segments=_N_NODES)
    pi = e / denom[rows]

    k_edges = jnp.asarray(_K_TABLE)[counts]
    zi = _sortable_keys(z)
    thre_key = _sc_thresholds(zi, rows, edge_start.astype(jnp.int32), k_edges)
    vk = lax.bitcast_convert_type(
        jnp.where(thre_key >= 0, thre_key, thre_key ^ jnp.int32(0x7FFFFFFF)),
        jnp.float32)
    thre = jnp.exp(vk - rmax) / denom

    combined = _combine_kernel(pi.T, thre[rows].T, gates[rows].T)
    return combined, loss
